# baseline (speedup 1.0000x reference)
import jax
import jax.numpy as jnp
from jax import lax
from jax.experimental import pallas as pl
from jax.experimental.pallas import tpu as pltpu
from jax.experimental.pallas import tpu_sc as plsc

_BATCH = 16384
_FEAT = 64
_NCLASS = 1000000
_NC = 2
_NS = 16
_NW = _NC * _NS
_CHUNK = 512
_NCHUNKS = 999936 // _CHUNK
_TAIL0 = 999936
_SCALE = 0.5 / _BATCH
_POSBITS = 14
_NSTAGE = 16


_NSPLIT = 4


def _fire_chunk(centersT_hbm, g, buf, sem):
    part = _CHUNK // _NSPLIT
    for u in range(_NSPLIT):
        pltpu.async_copy(
            centersT_hbm.at[:, pl.ds(
                pl.multiple_of(g * _CHUNK + u * part, 128), part)],
            buf.at[:, pl.ds(u * part, part)], sem)


def _sc_body(label_hbm, centersT_hbm, tail_hbm, out_hbm,
             labels_v, hits_v, clist_v, buf0, buf1, stage_v, tail_v,
             sem_misc, sem_out, sem0, sem1):
    w = lax.axis_index("s") * _NC + lax.axis_index("c")
    lo = w * _NCHUNKS // _NW
    hi = (w + 1) * _NCHUNKS // _NW
    base = lo * _CHUNK
    limit = jnp.where(w == _NW - 1, _NCLASS, hi * _CHUNK)

    cp_lab = pltpu.async_copy(label_hbm, labels_v, sem_misc)
    cp_tail = pltpu.async_copy(tail_hbm, tail_v, sem_misc)
    _fire_chunk(centersT_hbm, lo, buf0, sem0)
    cp_lab.wait()
    cp_tail.wait()

    iota = lax.iota(jnp.int32, 16)

    def scan_body(j, cnt):
        labv = labels_v[pl.ds(j * 16, 16)]
        m = (labv >= base) & (labv < limit)
        packed = lax.shift_left(labv - base, _POSBITS) | (iota + j * 16)
        plsc.store_compressed(hits_v.at[pl.ds(cnt, 16)], packed, mask=m)
        pc = plsc.all_reduce_population_count(m)
        return cnt + pc[0]

    cnt = lax.fori_loop(0, _BATCH // 16, scan_body, jnp.int32(0))
    cntv = jnp.broadcast_to(cnt, (16,))

    def emit_hits(buf, c0l, ccnt, nout0):
        def hit(e, nout):
            pk = clist_v[pl.ds(e, 16)][0]
            pos = pk & (2 ** _POSBITS - 1)
            col = lax.shift_right_logical(pk, _POSBITS) - c0l
            colv = jnp.broadcast_to(col, (16,))
            slot = nout % _NSTAGE
            soff = slot * _FEAT
            for q in range(4):
                cv = plsc.load_gather(buf, [iota + q * 16, colv])
                stage_v[pl.ds(soff + q * 16, 16)] = cv

            @pl.when(nout >= _NSTAGE)
            def _():
                pltpu.make_async_copy(
                    out_hbm.at[pl.ds(0, _FEAT)],
                    stage_v.at[pl.ds(0, _FEAT)], sem_out).wait()

            pltpu.async_copy(
                stage_v.at[pl.ds(soff, _FEAT)],
                out_hbm.at[pl.ds(pos * _FEAT, _FEAT)], sem_out)
            return nout + 1

        return lax.fori_loop(0, ccnt, hit, nout0)

    def rescan_range(lo_l, hi_l):
        def rescan(j, ccnt):
            pv = hits_v[pl.ds(j * 16, 16)]
            labl = lax.shift_right_logical(pv, _POSBITS)
            m = ((iota + j * 16) < cntv) & (labl >= lo_l) & (labl < hi_l)
            plsc.store_compressed(clist_v.at[pl.ds(ccnt, 16)], pv, mask=m)
            pc = plsc.all_reduce_population_count(m)
            return ccnt + pc[0]

        return lax.fori_loop(0, (cnt + 15) // 16, rescan, jnp.int32(0))

    def process_chunk(g, buf, nout):
        c0l = g * _CHUNK - base
        ccnt = rescan_range(c0l, c0l + _CHUNK)
        return emit_hits(buf, c0l, ccnt, nout)

    npairs = (hi - lo + 1) // 2

    def pair_body(p, nout):
        g0 = lo + 2 * p
        g1 = g0 + 1
        pltpu.make_async_copy(
            centersT_hbm.at[:, pl.ds(0, _CHUNK)], buf0, sem0).wait()

        @pl.when(g1 < hi)
        def _():
            _fire_chunk(centersT_hbm, g1, buf1, sem1)

        nout = process_chunk(g0, buf0, nout)

        def odd(nout_):
            pltpu.make_async_copy(
                centersT_hbm.at[:, pl.ds(0, _CHUNK)], buf1, sem1).wait()

            @pl.when(g1 + 1 < hi)
            def _():
                _fire_chunk(centersT_hbm, g1 + 1, buf0, sem0)

            return process_chunk(g1, buf1, nout_)

        return lax.cond(g1 < hi, odd, lambda n: n, nout)

    nout = lax.fori_loop(0, npairs, pair_body, jnp.int32(0))

    def tail_hits(nout_):
        c0l = _TAIL0 - base
        ccnt = rescan_range(c0l, c0l + 2 ** (31 - _POSBITS))
        return emit_hits(tail_v, c0l, ccnt, nout_)

    nout = lax.cond(w == _NW - 1, tail_hits, lambda n: n, nout)

    def drain(_, __):
        pltpu.make_async_copy(
            out_hbm.at[pl.ds(0, _FEAT)],
            stage_v.at[pl.ds(0, _FEAT)], sem_out).wait()
        return __

    lax.fori_loop(0, jnp.minimum(nout, _NSTAGE), drain, jnp.int32(0))


def _finish_body(feat_ref, g_ref, out_ref):
    d = feat_ref[...] - g_ref[...]
    out_ref[0, 0] = jnp.sum(d * d) * _SCALE


@jax.jit
def kernel(label, feat, centers):
    centersT = centers.T
    tail = lax.slice(centersT, (0, _TAIL0), (_FEAT, _NCLASS))

    sc = pl.kernel(
        _sc_body,
        out_type=jax.ShapeDtypeStruct((_BATCH * _FEAT,), jnp.float32),
        mesh=plsc.VectorSubcoreMesh(core_axis_name="c", subcore_axis_name="s"),
        compiler_params=pltpu.CompilerParams(needs_layout_passes=False),
        scratch_types=[
            pltpu.VMEM((_BATCH,), jnp.int32),
            pltpu.VMEM((_BATCH + 16,), jnp.int32),
            pltpu.VMEM((_BATCH + 16,), jnp.int32),
            pltpu.VMEM((_FEAT, _CHUNK), jnp.float32),
            pltpu.VMEM((_FEAT, _CHUNK), jnp.float32),
            pltpu.VMEM((_NSTAGE * _FEAT,), jnp.float32),
            pltpu.VMEM((_FEAT, _FEAT), jnp.float32),
            pltpu.SemaphoreType.DMA,
            pltpu.SemaphoreType.DMA,
            pltpu.SemaphoreType.DMA,
            pltpu.SemaphoreType.DMA,
        ],
    )
    gathered = sc(label, centersT, tail)

    loss11 = pl.pallas_call(
        _finish_body,
        out_shape=jax.ShapeDtypeStruct((1, 1), jnp.float32),
        out_specs=pl.BlockSpec(memory_space=pltpu.SMEM),
    )(feat, gathered.reshape(_BATCH, _FEAT))
    return loss11[0, 0]


# speedup vs baseline: 1.0845x; 1.0845x over previous
import jax
import jax.numpy as jnp
from jax import lax
from jax.experimental import pallas as pl
from jax.experimental.pallas import tpu as pltpu
from jax.experimental.pallas import tpu_sc as plsc

_BATCH = 16384
_FEAT = 64
_NCLASS = 1000000
_NC = 2
_NS = 16
_NW = _NC * _NS
_CHUNK = 512
_NCHUNKS = 999936 // _CHUNK
_TAIL0 = 999936
_SCALE = 0.5 / _BATCH
_POSBITS = 14
_NSTAGE = 16


_NSPLIT = 1


def _fire_chunk(centersT_hbm, g, buf, sem):
    part = _CHUNK // _NSPLIT
    for u in range(_NSPLIT):
        pltpu.async_copy(
            centersT_hbm.at[:, pl.ds(
                pl.multiple_of(g * _CHUNK + u * part, 128), part)],
            buf.at[:, pl.ds(u * part, part)], sem)


def _sc_body(label_hbm, centersT_hbm, tail_hbm, out_hbm,
             labels_v, hits_v, clist_v, buf0, buf1, stage_v, tail_v,
             sem_misc, sem_out, sem0, sem1):
    w = lax.axis_index("s") * _NC + lax.axis_index("c")
    lo = w * _NCHUNKS // _NW
    hi = (w + 1) * _NCHUNKS // _NW
    base = lo * _CHUNK
    limit = jnp.where(w == _NW - 1, _NCLASS, hi * _CHUNK)

    cp_lab = pltpu.async_copy(label_hbm, labels_v, sem_misc)
    cp_tail = pltpu.async_copy(tail_hbm, tail_v, sem_misc)
    _fire_chunk(centersT_hbm, lo, buf0, sem0)

    @pl.when(lo + 1 < hi)
    def _():
        _fire_chunk(centersT_hbm, lo + 1, buf1, sem1)

    cp_lab.wait()
    cp_tail.wait()

    iota = lax.iota(jnp.int32, 16)

    def scan_body(j, cnt):
        labv = labels_v[pl.ds(j * 16, 16)]
        m = (labv >= base) & (labv < limit)
        packed = lax.shift_left(labv - base, _POSBITS) | (iota + j * 16)
        plsc.store_compressed(hits_v.at[pl.ds(cnt, 16)], packed, mask=m)
        pc = plsc.all_reduce_population_count(m)
        return cnt + pc[0]

    cnt = lax.fori_loop(0, _BATCH // 16, scan_body, jnp.int32(0))
    cntv = jnp.broadcast_to(cnt, (16,))

    def emit_hits(buf, c0l, ccnt, nout0):
        def hit(e, nout):
            pk = clist_v[pl.ds(e, 16)][0]
            pos = pk & (2 ** _POSBITS - 1)
            col = lax.shift_right_logical(pk, _POSBITS) - c0l
            colv = jnp.broadcast_to(col, (16,))
            slot = nout % _NSTAGE
            soff = slot * _FEAT
            for q in range(4):
                cv = plsc.load_gather(buf, [iota + q * 16, colv])
                stage_v[pl.ds(soff + q * 16, 16)] = cv

            @pl.when(nout >= _NSTAGE)
            def _():
                pltpu.make_async_copy(
                    out_hbm.at[pl.ds(0, _FEAT)],
                    stage_v.at[pl.ds(0, _FEAT)], sem_out).wait()

            pltpu.async_copy(
                stage_v.at[pl.ds(soff, _FEAT)],
                out_hbm.at[pl.ds(pos * _FEAT, _FEAT)], sem_out)
            return nout + 1

        return lax.fori_loop(0, ccnt, hit, nout0)

    def rescan_range(lo_l, hi_l):
        def rescan(j, ccnt):
            pv = hits_v[pl.ds(j * 16, 16)]
            labl = lax.shift_right_logical(pv, _POSBITS)
            m = ((iota + j * 16) < cntv) & (labl >= lo_l) & (labl < hi_l)
            plsc.store_compressed(clist_v.at[pl.ds(ccnt, 16)], pv, mask=m)
            pc = plsc.all_reduce_population_count(m)
            return ccnt + pc[0]

        return lax.fori_loop(0, (cnt + 15) // 16, rescan, jnp.int32(0))

    def process_chunk(g, buf, nout):
        c0l = g * _CHUNK - base
        ccnt = rescan_range(c0l, c0l + _CHUNK)
        return emit_hits(buf, c0l, ccnt, nout)

    npairs = (hi - lo + 1) // 2

    def pair_body(p, nout):
        g0 = lo + 2 * p
        g1 = g0 + 1
        pltpu.make_async_copy(
            centersT_hbm.at[:, pl.ds(0, _CHUNK)], buf0, sem0).wait()

        nout = process_chunk(g0, buf0, nout)

        @pl.when(g0 + 2 < hi)
        def _():
            _fire_chunk(centersT_hbm, g0 + 2, buf0, sem0)

        def odd(nout_):
            pltpu.make_async_copy(
                centersT_hbm.at[:, pl.ds(0, _CHUNK)], buf1, sem1).wait()

            nout_ = process_chunk(g1, buf1, nout_)

            @pl.when(g1 + 2 < hi)
            def _():
                _fire_chunk(centersT_hbm, g1 + 2, buf1, sem1)

            return nout_

        return lax.cond(g1 < hi, odd, lambda n: n, nout)

    nout = lax.fori_loop(0, npairs, pair_body, jnp.int32(0))

    def tail_hits(nout_):
        c0l = _TAIL0 - base
        ccnt = rescan_range(c0l, c0l + 2 ** (31 - _POSBITS))
        return emit_hits(tail_v, c0l, ccnt, nout_)

    nout = lax.cond(w == _NW - 1, tail_hits, lambda n: n, nout)

    def drain(_, __):
        pltpu.make_async_copy(
            out_hbm.at[pl.ds(0, _FEAT)],
            stage_v.at[pl.ds(0, _FEAT)], sem_out).wait()
        return __

    lax.fori_loop(0, jnp.minimum(nout, _NSTAGE), drain, jnp.int32(0))


def _finish_body(feat_ref, g_ref, out_ref):
    d = feat_ref[...] - g_ref[...]
    out_ref[0, 0] = jnp.sum(d * d) * _SCALE


@jax.jit
def kernel(label, feat, centers):
    centersT = centers.T
    tail = lax.slice(centersT, (0, _TAIL0), (_FEAT, _NCLASS))

    sc = pl.kernel(
        _sc_body,
        out_type=jax.ShapeDtypeStruct((_BATCH * _FEAT,), jnp.float32),
        mesh=plsc.VectorSubcoreMesh(core_axis_name="c", subcore_axis_name="s"),
        compiler_params=pltpu.CompilerParams(needs_layout_passes=False),
        scratch_types=[
            pltpu.VMEM((_BATCH,), jnp.int32),
            pltpu.VMEM((_BATCH + 16,), jnp.int32),
            pltpu.VMEM((_BATCH + 16,), jnp.int32),
            pltpu.VMEM((_FEAT, _CHUNK), jnp.float32),
            pltpu.VMEM((_FEAT, _CHUNK), jnp.float32),
            pltpu.VMEM((_NSTAGE * _FEAT,), jnp.float32),
            pltpu.VMEM((_FEAT, _FEAT), jnp.float32),
            pltpu.SemaphoreType.DMA,
            pltpu.SemaphoreType.DMA,
            pltpu.SemaphoreType.DMA,
            pltpu.SemaphoreType.DMA,
        ],
    )
    gathered = sc(label, centersT, tail)

    loss11 = pl.pallas_call(
        _finish_body,
        out_shape=jax.ShapeDtypeStruct((1, 1), jnp.float32),
        out_specs=pl.BlockSpec(memory_space=pltpu.SMEM),
    )(feat, gathered.reshape(_BATCH, _FEAT))
    return loss11[0, 0]
